# baseline (speedup 1.0000x reference)
import functools

import jax
import jax.numpy as jnp
from jax import lax
from jax.experimental import pallas as pl
from jax.experimental.pallas import tpu as pltpu
from jax.experimental.pallas import tpu_sc as plsc

D = 32
L = 16
IDX_CHUNK = 128


@functools.lru_cache(maxsize=None)
def _build_sc_kernel(b: int, nc: int, ns: int):
    nw = nc * ns
    b_per_w = b // nw
    n_chunks = b_per_w // L
    g = b_per_w // IDX_CHUNK
    mesh = plsc.VectorSubcoreMesh(core_axis_name="c", subcore_axis_name="s")

    @functools.partial(
        pl.kernel,
        mesh=mesh,
        out_type=jax.ShapeDtypeStruct((nw, L), jnp.float32),
        compiler_params=pltpu.CompilerParams(needs_layout_passes=False,
                                             use_tc_tiling_on_sc=False),
        scratch_types=[
            pltpu.VMEM((g, IDX_CHUNK), jnp.int32),
            pltpu.VMEM((g, IDX_CHUNK), jnp.int32),
            pltpu.VMEM((b_per_w,), jnp.float32),
            pltpu.VMEM((b_per_w, D), jnp.float32),
            pltpu.VMEM((b_per_w, D), jnp.float32),
            pltpu.VMEM((L,), jnp.float32),
            pltpu.SemaphoreType.DMA,
            pltpu.SemaphoreType.DMA,
        ],
    )
    def sc_kernel(item_idx_hbm, ctx_idx_hbm, sppmi_hbm, items_hbm, ctxs_hbm,
                  out_hbm, iidx_v, cidx_v, sppmi_v, irows_v, crows_v, res_v,
                  sem_a, sem_b):
        wid = lax.axis_index("s") * nc + lax.axis_index("c")

        pltpu.sync_copy(item_idx_hbm.at[wid], iidx_v)
        pltpu.sync_copy(ctx_idx_hbm.at[wid], cidx_v)

        copies = []
        for j in range(g):
            rows = pl.ds(j * IDX_CHUNK, IDX_CHUNK)
            copies.append(
                pltpu.async_copy(items_hbm.at[iidx_v.at[j]], irows_v.at[rows],
                                 sem_a))
            copies.append(
                pltpu.async_copy(ctxs_hbm.at[cidx_v.at[j]], crows_v.at[rows],
                                 sem_b))
        pltpu.sync_copy(sppmi_hbm.at[wid], sppmi_v)
        for cp in copies:
            cp.wait()

        lane = lax.iota(jnp.int32, L)

        def chunk_body(t, acc):
            row_idx = t * L + lane
            pred = jnp.zeros((L,), jnp.float32)
            for d in range(D):
                col = jnp.full((L,), d, jnp.int32)
                a = plsc.load_gather(irows_v, [row_idx, col])
                c = plsc.load_gather(crows_v, [row_idx, col])
                pred = pred + a * c
            s = sppmi_v[pl.ds(t * L, L)]
            diff = s - pred
            return acc + diff * diff

        acc = lax.fori_loop(0, n_chunks, chunk_body,
                            jnp.zeros((L,), jnp.float32))
        res_v[...] = acc
        pltpu.sync_copy(res_v, out_hbm.at[wid])

    return sc_kernel


def kernel(user_id, item_id, rating, users, items, contexts):
    del users
    b = user_id.shape[0]
    info = plsc.get_sparse_core_info()
    nc, ns = info.num_cores, info.num_subcores
    nw = nc * ns
    item_idx = user_id.astype(jnp.int32).reshape(nw, b // nw // IDX_CHUNK,
                                                 IDX_CHUNK)
    ctx_idx = item_id.astype(jnp.int32).reshape(nw, b // nw // IDX_CHUNK,
                                                IDX_CHUNK)
    sppmi = rating.astype(jnp.float32).reshape(nw, b // nw)
    partial = _build_sc_kernel(b, nc, ns)(item_idx, ctx_idx, sppmi, items,
                                          contexts)
    return jnp.sum(partial) / b
